# Initial kernel scaffold; baseline (speedup 1.0000x reference)
import jax
import jax.numpy as jnp
from jax.experimental import pallas as pl


def kernel(x, W):
    raise NotImplementedError("write your pallas kernel here")


# speedup vs baseline: 22.1484x; 22.1484x over previous
import functools
import math

import jax
import jax.numpy as jnp
import numpy as np
from jax.experimental import pallas as pl

_THRESHOLD = 0.9
_ALPHA = 0.01
_NEG = -1e9


def _mono_u32_const(f):
    u = np.float32(f).view(np.uint32)
    m = np.uint32(0xFFFFFFFF) if (u >> np.uint32(31)) else np.uint32(0x80000000)
    return np.uint32(u ^ m)


_NEG_KEY = _mono_u32_const(_NEG)


def _router_block_kernel(x_ref, w_ref, p_ref, key_ref):
    xb = x_ref[...]
    wb = w_ref[...]
    logits = jax.lax.dot_general(
        xb, wb, (((1,), (1,)), ((), ())),
        preferred_element_type=jnp.float32)
    m = jnp.max(logits, axis=1, keepdims=True)
    ex = jnp.exp(logits - m)
    p = ex / jnp.sum(ex, axis=1, keepdims=True)
    E = p.shape[1]
    pa = p[:, :, None]
    pb = p[:, None, :]
    tie = (jax.lax.broadcasted_iota(jnp.int32, (E, E), 0)
           < jax.lax.broadcasted_iota(jnp.int32, (E, E), 1))[None]
    G = (pa > pb) | ((pa == pb) & tie)
    rank = jnp.sum(G.astype(jnp.int32), axis=1)
    s_excl = jnp.sum(jnp.where(G, pa, 0.0), axis=1)
    assigned = s_excl < _THRESHOLD
    r = jnp.where(assigned, p - (rank + 1).astype(jnp.float32),
                  jnp.float32(_NEG))
    u = jax.lax.bitcast_convert_type(r, jnp.uint32)
    mask = jnp.where((u >> 31) > 0, jnp.uint32(0xFFFFFFFF),
                     jnp.uint32(0x80000000))
    p_ref[...] = p
    key_ref[...] = u ^ mask


def _select_kernel(p_ref, key_ref, mask_ref, scores_ref, top1_ref, aux_ref,
                   *, cap):
    ku = key_ref[...]
    p = p_ref[...]
    N, E = p.shape
    C = cap
    v = jnp.zeros((1, E), jnp.uint32)
    for i in range(32):
        bit = np.uint32(1 << (31 - i))
        cand = v | bit
        cnt = jnp.sum((ku >= cand).astype(jnp.int32), axis=0, keepdims=True)
        v = jnp.where(cnt >= C, cand, v)
    cnt_gt = jnp.sum((ku > v).astype(jnp.int32), axis=0, keepdims=True)
    need = C - cnt_gt
    eq = (ku == v).astype(jnp.int32)
    s = eq
    sh = 1
    while sh < N:
        s = s + jnp.concatenate(
            [jnp.zeros((sh, E), jnp.int32), s[:N - sh]], axis=0)
        sh *= 2
    excl = s - eq
    keep = (ku > v) | ((ku == v) & (excl < need))
    finalm = keep & (ku > _NEG_KEY)
    mask_ref[...] = finalm.astype(jnp.int8)
    scores_ref[...] = jnp.where(finalm, p, 0.0)
    mx = jnp.max(p, axis=1, keepdims=True)
    lane = jax.lax.broadcasted_iota(jnp.int32, (N, E), 1)
    idx = jnp.min(jnp.where(p == mx, lane, E), axis=1, keepdims=True)
    top1_ref[...] = idx
    onehot = lane == idx
    fi = jnp.sum(onehot.astype(jnp.float32), axis=0, keepdims=True) / N
    pi = jnp.sum(jnp.where(onehot, mx, 0.0), axis=0, keepdims=True) / N
    aux_ref[...] = jnp.sum(fi * pi, axis=1, keepdims=True) * (E * _ALPHA)


def kernel(x, W):
    B, T, H = x.shape
    N = B * T
    E = W.shape[0]
    C = min(int(math.ceil(N / E) * 1.0), N)
    M = 128
    x_flat = x.reshape(N, H)
    p, ku = pl.pallas_call(
        _router_block_kernel,
        grid=(N // M,),
        in_specs=[pl.BlockSpec((M, H), lambda i: (i, 0)),
                  pl.BlockSpec((E, H), lambda i: (0, 0))],
        out_specs=[pl.BlockSpec((M, E), lambda i: (i, 0)),
                   pl.BlockSpec((M, E), lambda i: (i, 0))],
        out_shape=[jax.ShapeDtypeStruct((N, E), jnp.float32),
                   jax.ShapeDtypeStruct((N, E), jnp.uint32)],
    )(x_flat, W)
    maski8, scores, top1, aux = pl.pallas_call(
        functools.partial(_select_kernel, cap=C),
        out_shape=[jax.ShapeDtypeStruct((N, E), jnp.int8),
                   jax.ShapeDtypeStruct((N, E), jnp.float32),
                   jax.ShapeDtypeStruct((N, 1), jnp.int32),
                   jax.ShapeDtypeStruct((1, 1), jnp.float32)],
    )(p, ku)
    return (maski8.astype(bool), scores, aux[0, 0], top1.reshape(N))
